# baseline (speedup 1.0000x reference)
import functools

import jax
import jax.numpy as jnp
from jax import lax
from jax.experimental import pallas as pl
from jax.experimental.pallas import tpu as pltpu
from jax.experimental.pallas import tpu_sc as plsc

N_NODES = 10000
NP = 10240
N_EDGES = 160000
D = 256
DH = 128
N_TILES = 16
E_PER_TILE = N_EDGES // N_TILES
CHUNK = 80
N_CHUNKS = E_PER_TILE // CHUNK
N_SB = 5
SB_CHUNKS = N_CHUNKS // N_SB
ROWS_PER_TILE = NP // N_TILES


def _sc_body(x_hbm, dst_hbm, src_hbm,
             s0_hbm, s1_hbm, cnt_hbm,
             dst_v, src_v, gbuf0, gbuf1, ones_v, zrow_v, czero_v,
             accum, cnt_sp, sem0, sem1):
  cid = lax.axis_index("c")
  sid = lax.axis_index("s")

  @pl.loop(0, 8)
  def _zrow(i):
    zrow_v[i, pl.ds(0, 16)] = jnp.zeros((16,), jnp.float32)
    zrow_v[i, pl.ds(16, 16)] = jnp.zeros((16,), jnp.float32)
    zrow_v[i, pl.ds(32, 16)] = jnp.zeros((16,), jnp.float32)
    zrow_v[i, pl.ds(48, 16)] = jnp.zeros((16,), jnp.float32)
    zrow_v[i, pl.ds(64, 16)] = jnp.zeros((16,), jnp.float32)
    zrow_v[i, pl.ds(80, 16)] = jnp.zeros((16,), jnp.float32)
    zrow_v[i, pl.ds(96, 16)] = jnp.zeros((16,), jnp.float32)
    zrow_v[i, pl.ds(112, 16)] = jnp.zeros((16,), jnp.float32)

  @pl.loop(0, ROWS_PER_TILE // 8)
  def _zacc(j):
    pltpu.sync_copy(zrow_v, accum.at[pl.ds(sid * ROWS_PER_TILE + j * 8, 8)])

  @pl.when(cid == 0)
  def _cnt_init():
    @pl.loop(0, CHUNK // 16)
    def _ones(i):
      ones_v[pl.ds(i * 16, 16)] = jnp.ones((16,), jnp.float32)

    @pl.when(sid == 0)
    def _czero():
      @pl.loop(0, 64)
      def _cz(i):
        czero_v[pl.ds(i * 16, 16)] = jnp.zeros((16,), jnp.float32)

      @pl.loop(0, NP // 1024)
      def _czs(j):
        pltpu.sync_copy(czero_v, cnt_sp.at[pl.ds(j * 1024, 1024)])

  plsc.subcore_barrier()

  def run_core(x_hbm, out_hbm, do_count):
    def gather_start(g, buf, sem):
      return pltpu.async_copy(x_hbm.at[src_v.at[g]], buf, sem)

    def consume(g, buf, sem):
      pltpu.make_async_copy(x_hbm.at[src_v.at[g]], buf, sem).wait()
      pltpu.sync_copy(buf, accum.at[dst_v.at[g]], add=True)
      if do_count:
        pltpu.sync_copy(ones_v, cnt_sp.at[dst_v.at[g]], add=True)

    @pl.loop(0, N_SB)
    def _sb(j):
      pltpu.sync_copy(dst_hbm.at[sid, j], dst_v)
      pltpu.sync_copy(src_hbm.at[sid, j], src_v)

      gather_start(0, gbuf0, sem0)

      @pl.loop(0, SB_CHUNKS - 1, step=2)
      def _main(g):
        gather_start(g + 1, gbuf1, sem1)
        consume(g, gbuf0, sem0)
        gather_start(g + 2, gbuf0, sem0)
        consume(g + 1, gbuf1, sem1)

      consume(SB_CHUNKS - 1, gbuf0, sem0)

    plsc.subcore_barrier()

    pltpu.sync_copy(accum.at[pl.ds(sid * ROWS_PER_TILE, ROWS_PER_TILE)],
                    out_hbm.at[pl.ds(sid * ROWS_PER_TILE, ROWS_PER_TILE)])
    if do_count:
      @pl.when(sid == 0)
      def _cnt_out():
        pltpu.sync_copy(cnt_sp, cnt_hbm)

  @pl.when(cid == 0)
  def _core0():
    run_core(x_hbm.at[:, pl.ds(0, DH)], s0_hbm, True)

  @pl.when(cid == 1)
  def _core1():
    run_core(x_hbm.at[:, pl.ds(DH, DH)], s1_hbm, False)


@jax.jit
def _sc_aggregate(x, dst_r, src_r):
  mesh = plsc.VectorSubcoreMesh(core_axis_name="c", subcore_axis_name="s")
  f = pl.kernel(
      _sc_body,
      out_type=[
          jax.ShapeDtypeStruct((NP, DH), jnp.float32),
          jax.ShapeDtypeStruct((NP, DH), jnp.float32),
          jax.ShapeDtypeStruct((NP,), jnp.float32),
      ],
      mesh=mesh,
      scratch_types=[
          pltpu.VMEM((SB_CHUNKS, CHUNK), jnp.int32),
          pltpu.VMEM((SB_CHUNKS, CHUNK), jnp.int32),
          pltpu.VMEM((CHUNK, DH), jnp.float32),
          pltpu.VMEM((CHUNK, DH), jnp.float32),
          pltpu.VMEM((CHUNK,), jnp.float32),
          pltpu.VMEM((8, DH), jnp.float32),
          pltpu.VMEM((1024,), jnp.float32),
          pltpu.VMEM_SHARED((NP, DH), jnp.float32),
          pltpu.VMEM_SHARED((NP,), jnp.float32),
          pltpu.SemaphoreType.DMA,
          pltpu.SemaphoreType.DMA,
      ],
  )
  return f(x, dst_r, src_r)


BN = 1000


def _xw_body(x_ref, wrt_ref, b_ref, o_ref):
  o_ref[...] = jnp.dot(x_ref[...], wrt_ref[...],
                       preferred_element_type=jnp.float32) + b_ref[...]


@jax.jit
def _tc_xw(x, wrt, b):
  return pl.pallas_call(
      _xw_body,
      grid=(N_NODES // BN,),
      in_specs=[
          pl.BlockSpec((BN, D), lambda i: (i, 0)),
          pl.BlockSpec((D, D), lambda i: (0, 0)),
          pl.BlockSpec((1, D), lambda i: (0, 0)),
      ],
      out_specs=pl.BlockSpec((BN, D), lambda i: (i, 0)),
      out_shape=jax.ShapeDtypeStruct((N_NODES, D), jnp.float32),
  )(x, wrt, b)


def _tc_body(s0_ref, s1_ref, cnt_ref, xw_ref, x_ref, wlt_ref, o_ref):
  recip = 1.0 / jnp.maximum(cnt_ref[...], 1.0)
  m0 = s0_ref[...] * recip
  m1 = s1_ref[...] * recip
  acc = jnp.dot(m0, wlt_ref[0:DH, :], preferred_element_type=jnp.float32)
  acc = acc + jnp.dot(m1, wlt_ref[DH:D, :], preferred_element_type=jnp.float32)
  acc = acc + xw_ref[...]
  o_ref[...] = x_ref[...] + jnp.maximum(acc, 0.0)


@jax.jit
def _tc_dense(s0, s1, cnt, xw, x, wlt):
  return pl.pallas_call(
      _tc_body,
      grid=(N_NODES // BN,),
      in_specs=[
          pl.BlockSpec((BN, DH), lambda i: (i, 0)),
          pl.BlockSpec((BN, DH), lambda i: (i, 0)),
          pl.BlockSpec((BN, 1), lambda i: (i, 0)),
          pl.BlockSpec((BN, D), lambda i: (i, 0)),
          pl.BlockSpec((BN, D), lambda i: (i, 0)),
          pl.BlockSpec((D, D), lambda i: (0, 0)),
      ],
      out_specs=pl.BlockSpec((BN, D), lambda i: (i, 0)),
      out_shape=jax.ShapeDtypeStruct((N_NODES, D), jnp.float32),
  )(s0, s1, cnt, xw, x, wlt)


def kernel(x, edge_index, W_l, b_l, W_r, b_r):
  ei = edge_index.astype(jnp.int32)
  dst = ei[0].reshape(N_TILES, N_SB, SB_CHUNKS, CHUNK)
  src = ei[1].reshape(N_TILES, N_SB, SB_CHUNKS, CHUNK)
  s0, s1, cnt = _sc_aggregate(x, dst, src)
  xw = _tc_xw(x, W_r.T, (b_l + b_r).reshape(1, D))
  return _tc_dense(s0, s1, cnt.reshape(NP, 1), xw, x, W_l.T)


# speedup vs baseline: 1.0183x; 1.0183x over previous
import jax
import jax.numpy as jnp
from jax import lax
from jax.experimental import pallas as pl
from jax.experimental.pallas import tpu as pltpu
from jax.experimental.pallas import tpu_sc as plsc

N_NODES = 10000
NP = 10240
N_EDGES = 160000
D = 256
DH = 128
N_TILES = 16
E_PER_TILE = N_EDGES // N_TILES
CHUNK = 80
N_CHUNKS = E_PER_TILE // CHUNK
N_SB = 5
SB_CHUNKS = N_CHUNKS // N_SB
ROWS_PER_TILE = NP // N_TILES


def _sc_body(x_hbm, dst_hbm, src_hbm,
             s0_hbm, s1_hbm, cnt0_hbm, cnt1_hbm,
             dstA, srcA, dstB, srcB, gbuf0, gbuf1, ones_v, zrow_v, czero_v,
             accum, cnt_sp, sem0, sem1, semA, semB):
  cid = lax.axis_index("c")
  sid = lax.axis_index("s")

  @pl.loop(0, 8)
  def _zrow(i):
    for c in range(8):
      zrow_v[i, pl.ds(c * 16, 16)] = jnp.zeros((16,), jnp.float32)

  @pl.loop(0, ROWS_PER_TILE // 8)
  def _zacc(j):
    pltpu.sync_copy(zrow_v, accum.at[pl.ds(sid * ROWS_PER_TILE + j * 8, 8)])

  @pl.loop(0, CHUNK // 16)
  def _ones(i):
    ones_v[pl.ds(i * 16, 16)] = jnp.ones((16,), jnp.float32)

  @pl.when(sid == 0)
  def _czero():
    @pl.loop(0, 64)
    def _cz(i):
      czero_v[pl.ds(i * 16, 16)] = jnp.zeros((16,), jnp.float32)

    @pl.loop(0, NP // 1024)
    def _czs(j):
      pltpu.sync_copy(czero_v, cnt_sp.at[pl.ds(j * 1024, 1024)])

  plsc.subcore_barrier()

  def run_core(x_view, out_hbm, cnt_out_hbm, count_parity):
    def idx_start(j, dbuf, sbuf, sem):
      pltpu.async_copy(dst_hbm.at[sid, j], dbuf, sem)
      pltpu.async_copy(src_hbm.at[sid, j], sbuf, sem)

    def idx_wait(j, dbuf, sbuf, sem):
      pltpu.make_async_copy(dst_hbm.at[sid, j], dbuf, sem).wait()
      pltpu.make_async_copy(src_hbm.at[sid, j], sbuf, sem).wait()

    def process_sb(dbuf, sbuf, do_count):
      def gather_start(g, buf, sem):
        pltpu.async_copy(x_view.at[sbuf.at[g]], buf, sem)

      def consume(g, buf, sem):
        pltpu.make_async_copy(x_view.at[sbuf.at[g]], buf, sem).wait()
        pltpu.sync_copy(buf, accum.at[dbuf.at[g]], add=True)
        if do_count:
          pltpu.sync_copy(ones_v, cnt_sp.at[dbuf.at[g]], add=True)

      gather_start(0, gbuf0, sem0)

      @pl.loop(0, SB_CHUNKS - 1, step=2)
      def _main(g):
        gather_start(g + 1, gbuf1, sem1)
        consume(g, gbuf0, sem0)
        gather_start(g + 2, gbuf0, sem0)
        consume(g + 1, gbuf1, sem1)

      consume(SB_CHUNKS - 1, gbuf0, sem0)

    idx_start(0, dstA, srcA, semA)
    idx_start(1, dstB, srcB, semB)

    @pl.loop(0, N_SB - 1, step=2)
    def _sb(j):
      idx_wait(j, dstA, srcA, semA)
      process_sb(dstA, srcA, count_parity == 0)

      @pl.when(j + 2 < N_SB)
      def _pfA():
        idx_start(j + 2, dstA, srcA, semA)

      idx_wait(j + 1, dstB, srcB, semB)
      process_sb(dstB, srcB, count_parity == 1)

      @pl.when(j + 3 < N_SB)
      def _pfB():
        idx_start(j + 3, dstB, srcB, semB)

    idx_wait(N_SB - 1, dstA, srcA, semA)
    process_sb(dstA, srcA, count_parity == 0)

    plsc.subcore_barrier()

    pltpu.sync_copy(accum.at[pl.ds(sid * ROWS_PER_TILE, ROWS_PER_TILE)],
                    out_hbm.at[pl.ds(sid * ROWS_PER_TILE, ROWS_PER_TILE)])

    @pl.when(sid == 0)
    def _cnt_out():
      pltpu.sync_copy(cnt_sp, cnt_out_hbm)

  @pl.when(cid == 0)
  def _core0():
    run_core(x_hbm.at[:, pl.ds(0, DH)], s0_hbm, cnt0_hbm, 1)

  @pl.when(cid == 1)
  def _core1():
    run_core(x_hbm.at[:, pl.ds(DH, DH)], s1_hbm, cnt1_hbm, 0)


@jax.jit
def _sc_aggregate(x, dst_r, src_r):
  mesh = plsc.VectorSubcoreMesh(core_axis_name="c", subcore_axis_name="s")
  f = pl.kernel(
      _sc_body,
      out_type=[
          jax.ShapeDtypeStruct((NP, DH), jnp.float32),
          jax.ShapeDtypeStruct((NP, DH), jnp.float32),
          jax.ShapeDtypeStruct((NP,), jnp.float32),
          jax.ShapeDtypeStruct((NP,), jnp.float32),
      ],
      mesh=mesh,
      scratch_types=[
          pltpu.VMEM((SB_CHUNKS, CHUNK), jnp.int32),
          pltpu.VMEM((SB_CHUNKS, CHUNK), jnp.int32),
          pltpu.VMEM((SB_CHUNKS, CHUNK), jnp.int32),
          pltpu.VMEM((SB_CHUNKS, CHUNK), jnp.int32),
          pltpu.VMEM((CHUNK, DH), jnp.float32),
          pltpu.VMEM((CHUNK, DH), jnp.float32),
          pltpu.VMEM((CHUNK,), jnp.float32),
          pltpu.VMEM((8, DH), jnp.float32),
          pltpu.VMEM((1024,), jnp.float32),
          pltpu.VMEM_SHARED((NP, DH), jnp.float32),
          pltpu.VMEM_SHARED((NP,), jnp.float32),
          pltpu.SemaphoreType.DMA,
          pltpu.SemaphoreType.DMA,
          pltpu.SemaphoreType.DMA,
          pltpu.SemaphoreType.DMA,
      ],
  )
  return f(x, dst_r, src_r)


BN = 1000


def _tc_body(s0_ref, s1_ref, c0_ref, c1_ref, x_ref, wlt_ref, wrt_ref, b_ref,
             o_ref):
  cnt = jnp.maximum(c0_ref[...] + c1_ref[...], 1.0)
  recip = 1.0 / cnt
  m0 = s0_ref[...] * recip
  m1 = s1_ref[...] * recip
  acc = jnp.dot(m0, wlt_ref[0:DH, :], preferred_element_type=jnp.float32)
  acc = acc + jnp.dot(m1, wlt_ref[DH:D, :], preferred_element_type=jnp.float32)
  acc = acc + jnp.dot(x_ref[...], wrt_ref[...], preferred_element_type=jnp.float32)
  acc = acc + b_ref[...]
  o_ref[...] = x_ref[...] + jnp.maximum(acc, 0.0)


@jax.jit
def _tc_dense(s0, s1, c0, c1, x, wlt, wrt, b):
  return pl.pallas_call(
      _tc_body,
      grid=(N_NODES // BN,),
      in_specs=[
          pl.BlockSpec((BN, DH), lambda i: (i, 0)),
          pl.BlockSpec((BN, DH), lambda i: (i, 0)),
          pl.BlockSpec((BN, 1), lambda i: (i, 0)),
          pl.BlockSpec((BN, 1), lambda i: (i, 0)),
          pl.BlockSpec((BN, D), lambda i: (i, 0)),
          pl.BlockSpec((D, D), lambda i: (0, 0)),
          pl.BlockSpec((D, D), lambda i: (0, 0)),
          pl.BlockSpec((1, D), lambda i: (0, 0)),
      ],
      out_specs=pl.BlockSpec((BN, D), lambda i: (i, 0)),
      out_shape=jax.ShapeDtypeStruct((N_NODES, D), jnp.float32),
  )(s0, s1, c0, c1, x, wlt, wrt, b)


def kernel(x, edge_index, W_l, b_l, W_r, b_r):
  ei = edge_index.astype(jnp.int32)
  dst = ei[0].reshape(N_TILES, N_SB, SB_CHUNKS, CHUNK)
  src = ei[1].reshape(N_TILES, N_SB, SB_CHUNKS, CHUNK)
  s0, s1, c0, c1 = _sc_aggregate(x, dst, src)
  bias = (b_l + b_r).reshape(1, D)
  return _tc_dense(s0, s1, c0.reshape(NP, 1), c1.reshape(NP, 1), x,
                   W_l.T, W_r.T, bias)
